# Initial kernel scaffold; baseline (speedup 1.0000x reference)
import jax
import jax.numpy as jnp
from jax.experimental import pallas as pl


def kernel(X, edge_index, edge_values, gpr_weights):
    raise NotImplementedError("write your pallas kernel here")


# speedup vs baseline: 1.9317x; 1.9317x over previous
import functools

import jax
import jax.numpy as jnp
from jax import lax
from jax.experimental import pallas as pl
from jax.experimental.pallas import tpu as pltpu
from jax.experimental.pallas import tpu_sc as plsc

N = 10000
NP = 10240
D = 128
DH = D // 2
E = 320000
HOPS = 10
NC = 2
NS = 16
CHUNK = 128
CHUNKS_PER_TILE = 160
EDGES_PER_TILE = CHUNKS_PER_TILE * CHUNK
E_PAD = EDGES_PER_TILE * NS
ROWS_PER_TILE = NP // NS
PIECE = 128
PIECES = ROWS_PER_TILE // PIECE
NV = DH // 16


def _body(x0, x1, colh, rowh, valh, gwh, out0, out1, h0, h1,
          col_res, row_res, val_res, gbuf, stage, stage2, gw_res, p_sh, sem):
    cid = lax.axis_index("c")
    sid = lax.axis_index("s")

    cbase = sid * CHUNKS_PER_TILE
    pltpu.sync_copy(colh.at[pl.ds(cbase, CHUNKS_PER_TILE)], col_res)
    pltpu.sync_copy(rowh.at[pl.ds(cbase, CHUNKS_PER_TILE)], row_res)
    pltpu.sync_copy(valh.at[pl.ds(cbase, CHUNKS_PER_TILE)], val_res)
    pltpu.sync_copy(gwh, gw_res)

    def run(xh, hh, outh):
        base = sid * ROWS_PER_TILE
        gvec = gw_res[...]

        g0 = gvec[0]
        for k in range(PIECES):
            r0 = base + k * PIECE
            pltpu.sync_copy(xh.at[pl.ds(r0, PIECE)], stage)
            pltpu.sync_copy(stage, hh.at[pl.ds(r0, PIECE)])

            def init_row(i, _):
                for j in range(NV):
                    sl = pl.ds(j * 16, 16)
                    stage2[i, sl] = g0 * stage[i, sl]
                return 0

            lax.fori_loop(0, PIECE, init_row, 0)
            pltpu.sync_copy(stage2, outh.at[pl.ds(r0, PIECE)])
        plsc.subcore_barrier()

        def hop(l, _):
            def zero_row(i, _):
                z = jnp.zeros((16,), jnp.float32)
                for j in range(NV):
                    stage[i, pl.ds(j * 16, 16)] = z
                return 0

            lax.fori_loop(0, PIECE, zero_row, 0)
            for k in range(PIECES):
                pltpu.sync_copy(stage, p_sh.at[pl.ds(base + k * PIECE, PIECE)])
            plsc.subcore_barrier()

            def do_chunk(c, _):
                pltpu.async_copy(hh.at[col_res.at[c]], gbuf, sem).wait()

                def scale_group(gg, _):
                    vv = val_res[c, pl.ds(gg * 16, 16)]
                    for t in range(16):
                        v = vv[t]
                        e = gg * 16 + t
                        for j in range(NV):
                            sl = pl.ds(j * 16, 16)
                            gbuf[e, sl] = gbuf[e, sl] * v
                    return 0

                lax.fori_loop(0, CHUNK // 16, scale_group, 0)
                pltpu.sync_copy(gbuf, p_sh.at[row_res.at[c]], add=True)
                return 0

            lax.fori_loop(0, CHUNKS_PER_TILE, do_chunk, 0)
            plsc.subcore_barrier()

            g = jnp.float32(0.0)
            for t in range(1, HOPS + 1):
                g = lax.select(l == t, gvec[t], g)
            for k in range(PIECES):
                r0 = base + k * PIECE
                pltpu.sync_copy(p_sh.at[pl.ds(r0, PIECE)], stage)
                pltpu.sync_copy(stage, hh.at[pl.ds(r0, PIECE)])
                pltpu.sync_copy(outh.at[pl.ds(r0, PIECE)], stage2)

                def acc_row(i, _):
                    for j in range(NV):
                        sl = pl.ds(j * 16, 16)
                        stage2[i, sl] = stage2[i, sl] + g * stage[i, sl]
                    return 0

                lax.fori_loop(0, PIECE, acc_row, 0)
                pltpu.sync_copy(stage2, outh.at[pl.ds(r0, PIECE)])
            plsc.subcore_barrier()
            return 0

        lax.fori_loop(1, HOPS + 1, hop, 0)

    @pl.when(cid == 0)
    def _():
        run(x0, h0, out0)

    @pl.when(cid == 1)
    def _():
        run(x1, h1, out1)


def _build():
    mesh = plsc.VectorSubcoreMesh(core_axis_name="c", subcore_axis_name="s")
    f32 = jnp.float32
    return pl.kernel(
        _body,
        out_type=(
            jax.ShapeDtypeStruct((NP, DH), f32),
            jax.ShapeDtypeStruct((NP, DH), f32),
            jax.ShapeDtypeStruct((NP, DH), f32),
            jax.ShapeDtypeStruct((NP, DH), f32),
        ),
        mesh=mesh,
        scratch_types=(
            pltpu.VMEM((CHUNKS_PER_TILE, CHUNK), jnp.int32),
            pltpu.VMEM((CHUNKS_PER_TILE, CHUNK), jnp.int32),
            pltpu.VMEM((CHUNKS_PER_TILE, CHUNK), f32),
            pltpu.VMEM((CHUNK, DH), f32),
            pltpu.VMEM((PIECE, DH), f32),
            pltpu.VMEM((PIECE, DH), f32),
            pltpu.VMEM((16,), f32),
            pltpu.VMEM_SHARED((NP, DH), f32),
            pltpu.SemaphoreType.DMA,
        ),
        compiler_params=pltpu.CompilerParams(use_tc_tiling_on_sc=False),
    )


def kernel(X, edge_index, edge_values, gpr_weights):
    row = edge_index[0]
    col = edge_index[1]
    pad = E_PAD - E
    shape2d = (NS * CHUNKS_PER_TILE, CHUNK)
    colp = jnp.concatenate([col, jnp.zeros((pad,), jnp.int32)]).reshape(shape2d)
    rowp = jnp.concatenate([row, jnp.zeros((pad,), jnp.int32)]).reshape(shape2d)
    valp = jnp.concatenate(
        [edge_values, jnp.zeros((pad,), jnp.float32)]).reshape(shape2d)
    gwp = jnp.zeros((16,), jnp.float32).at[:HOPS + 1].set(gpr_weights)
    xp = jnp.pad(X, ((0, NP - N), (0, 0)))
    x0 = xp[:, :DH]
    x1 = xp[:, DH:]
    out0, out1, _, _ = _build()(x0, x1, colp, rowp, valp, gwp)
    return jnp.concatenate([out0[:N], out1[:N]], axis=1)


# speedup vs baseline: 2.7639x; 1.4308x over previous
import functools

import jax
import jax.numpy as jnp
from jax import lax
from jax.experimental import pallas as pl
from jax.experimental.pallas import tpu as pltpu
from jax.experimental.pallas import tpu_sc as plsc

N = 10000
NP = 10240
D = 128
DH = D // 2
E = 320000
HOPS = 10
NC = 2
NS = 16
CHUNK = 128
CHUNKS_PER_TILE = 160
EDGES_PER_TILE = CHUNKS_PER_TILE * CHUNK
E_PAD = EDGES_PER_TILE * NS
ROWS_PER_TILE = NP // NS
PIECE = 128
PIECES = ROWS_PER_TILE // PIECE
NV = DH // 16
PAIRS = CHUNKS_PER_TILE // 2


def _body(x0, x1, colh, rowh, valh, gwh, out0, out1, h0, h1,
          col_res, row_res, val_res, gbuf0, gbuf1, gw_res, p_sh,
          gsem0, gsem1, ssem0, ssem1):
    cid = lax.axis_index("c")
    sid = lax.axis_index("s")

    cbase = sid * CHUNKS_PER_TILE
    pltpu.sync_copy(colh.at[pl.ds(cbase, CHUNKS_PER_TILE)], col_res)
    pltpu.sync_copy(rowh.at[pl.ds(cbase, CHUNKS_PER_TILE)], row_res)
    pltpu.sync_copy(valh.at[pl.ds(cbase, CHUNKS_PER_TILE)], val_res)
    pltpu.sync_copy(gwh, gw_res)

    def run(xh, hh, outh):
        base = sid * ROWS_PER_TILE
        gvec = gw_res[...]

        def scale(buf, c):
            def scale_group(gg, _):
                vv = val_res[c, pl.ds(gg * 16, 16)]
                for t in range(16):
                    v = vv[t]
                    e = gg * 16 + t
                    for j in range(NV):
                        sl = pl.ds(j * 16, 16)
                        buf[e, sl] = buf[e, sl] * v
                return 0

            lax.fori_loop(0, CHUNK // 16, scale_group, 0)

        def start_gather(c, buf, sem):
            pltpu.async_copy(hh.at[col_res.at[c]], buf, sem)

        def wait_gather(buf, sem):
            pltpu.make_async_copy(hh.at[col_res.at[0]], buf, sem).wait()

        def start_scatter(c, buf, sem):
            pltpu.async_copy(buf, p_sh.at[row_res.at[c]], sem, add=True)

        def wait_scatter(buf, sem):
            pltpu.make_async_copy(buf, p_sh.at[row_res.at[0]], sem).wait()

        g0 = gvec[0]
        for k in range(PIECES):
            r0 = base + k * PIECE
            pltpu.sync_copy(xh.at[pl.ds(r0, PIECE)], gbuf0)
            pltpu.sync_copy(gbuf0, hh.at[pl.ds(r0, PIECE)])

            def init_row(i, _):
                for j in range(NV):
                    sl = pl.ds(j * 16, 16)
                    gbuf1[i, sl] = g0 * gbuf0[i, sl]
                return 0

            lax.fori_loop(0, PIECE, init_row, 0)
            pltpu.sync_copy(gbuf1, outh.at[pl.ds(r0, PIECE)])
        plsc.subcore_barrier()

        def hop(l, _):
            def zero_row(i, _):
                z = jnp.zeros((16,), jnp.float32)
                for j in range(NV):
                    gbuf0[i, pl.ds(j * 16, 16)] = z
                return 0

            lax.fori_loop(0, PIECE, zero_row, 0)
            for k in range(PIECES):
                pltpu.sync_copy(gbuf0, p_sh.at[pl.ds(base + k * PIECE, PIECE)])
            plsc.subcore_barrier()

            start_gather(0, gbuf0, gsem0)

            def pair(i, _):
                c0 = i * 2
                wait_gather(gbuf0, gsem0)

                @pl.when(i > 0)
                def _():
                    wait_scatter(gbuf1, ssem1)

                start_gather(c0 + 1, gbuf1, gsem1)
                scale(gbuf0, c0)
                start_scatter(c0, gbuf0, ssem0)

                wait_gather(gbuf1, gsem1)
                wait_scatter(gbuf0, ssem0)
                start_gather(c0 + 2, gbuf0, gsem0)
                scale(gbuf1, c0 + 1)
                start_scatter(c0 + 1, gbuf1, ssem1)
                return 0

            lax.fori_loop(0, PAIRS - 1, pair, 0)
            c0 = CHUNKS_PER_TILE - 2
            wait_gather(gbuf0, gsem0)
            wait_scatter(gbuf1, ssem1)
            start_gather(c0 + 1, gbuf1, gsem1)
            scale(gbuf0, c0)
            start_scatter(c0, gbuf0, ssem0)
            wait_gather(gbuf1, gsem1)
            scale(gbuf1, c0 + 1)
            wait_scatter(gbuf0, ssem0)
            start_scatter(c0 + 1, gbuf1, ssem1)
            wait_scatter(gbuf1, ssem1)
            plsc.subcore_barrier()

            g = jnp.float32(0.0)
            for t in range(1, HOPS + 1):
                g = lax.select(l == t, gvec[t], g)
            for k in range(PIECES):
                r0 = base + k * PIECE
                pltpu.sync_copy(p_sh.at[pl.ds(r0, PIECE)], gbuf0)
                pltpu.sync_copy(gbuf0, hh.at[pl.ds(r0, PIECE)])
                pltpu.sync_copy(outh.at[pl.ds(r0, PIECE)], gbuf1)

                def acc_row(i, _):
                    for j in range(NV):
                        sl = pl.ds(j * 16, 16)
                        gbuf1[i, sl] = gbuf1[i, sl] + g * gbuf0[i, sl]
                    return 0

                lax.fori_loop(0, PIECE, acc_row, 0)
                pltpu.sync_copy(gbuf1, outh.at[pl.ds(r0, PIECE)])
            plsc.subcore_barrier()
            return 0

        lax.fori_loop(1, HOPS + 1, hop, 0)

    @pl.when(cid == 0)
    def _():
        run(x0, h0, out0)

    @pl.when(cid == 1)
    def _():
        run(x1, h1, out1)


def _build():
    mesh = plsc.VectorSubcoreMesh(core_axis_name="c", subcore_axis_name="s")
    f32 = jnp.float32
    return pl.kernel(
        _body,
        out_type=(
            jax.ShapeDtypeStruct((NP, DH), f32),
            jax.ShapeDtypeStruct((NP, DH), f32),
            jax.ShapeDtypeStruct((NP, DH), f32),
            jax.ShapeDtypeStruct((NP, DH), f32),
        ),
        mesh=mesh,
        scratch_types=(
            pltpu.VMEM((CHUNKS_PER_TILE, CHUNK), jnp.int32),
            pltpu.VMEM((CHUNKS_PER_TILE, CHUNK), jnp.int32),
            pltpu.VMEM((CHUNKS_PER_TILE, CHUNK), f32),
            pltpu.VMEM((CHUNK, DH), f32),
            pltpu.VMEM((CHUNK, DH), f32),
            pltpu.VMEM((16,), f32),
            pltpu.VMEM_SHARED((NP, DH), f32),
            pltpu.SemaphoreType.DMA,
            pltpu.SemaphoreType.DMA,
            pltpu.SemaphoreType.DMA,
            pltpu.SemaphoreType.DMA,
        ),
        compiler_params=pltpu.CompilerParams(use_tc_tiling_on_sc=False),
    )


def kernel(X, edge_index, edge_values, gpr_weights):
    row = edge_index[0]
    col = edge_index[1]
    pad = E_PAD - E
    shape2d = (NS * CHUNKS_PER_TILE, CHUNK)
    colp = jnp.concatenate([col, jnp.zeros((pad,), jnp.int32)]).reshape(shape2d)
    rowp = jnp.concatenate([row, jnp.zeros((pad,), jnp.int32)]).reshape(shape2d)
    valp = jnp.concatenate(
        [edge_values, jnp.zeros((pad,), jnp.float32)]).reshape(shape2d)
    gwp = jnp.zeros((16,), jnp.float32).at[:HOPS + 1].set(gpr_weights)
    xp = jnp.pad(X, ((0, NP - N), (0, 0)))
    x0 = xp[:, :DH]
    x1 = xp[:, DH:]
    out0, out1, _, _ = _build()(x0, x1, colp, rowp, valp, gwp)
    return jnp.concatenate([out0[:N], out1[:N]], axis=1)
